# baseline (speedup 1.0000x reference)
import jax
import jax.numpy as jnp
from jax.experimental import pallas as pl
from jax.experimental.pallas import tpu as pltpu

F32 = jnp.float32
BF16 = jnp.bfloat16

D = 128
KEIG = 256
NCLS = 9
N0, N1, N2 = 2048, 6144, 4096

BP1 = 256
BP2 = 256
NP1 = N0 // BP1
NP2 = N1 // BP2


def _dot(a, b):
    return jax.lax.dot_general(a, b, (((1,), (0,)), ((), ())),
                               preferred_element_type=F32)


def _dot_tn(a, b):
    return jax.lax.dot_general(a, b, (((0,), (0,)), ((), ())),
                               preferred_element_type=F32)


def _net_body(x0, x1, x2, e0, ed1, eu1, evs,
              w0, w10, w1id, w1d, w1u, w01, w21, wout, bout,
              inc1, inc2,
              out,
              xw01, xw10, xw21, y0m, y1acc, x0n, s0s, s1s, y0mb,
              r1a, r1b, r2a, r2b, sem1, sem2):
    ring1 = (r1a, r1b)
    ring2 = (r2a, r2b)

    def cp1(i, slot):
        return pltpu.make_async_copy(
            inc1.at[pl.ds(i * BP1, BP1), :], ring1[slot], sem1.at[slot])

    def cp2(i, slot):
        return pltpu.make_async_copy(
            inc2.at[pl.ds(i * BP2, BP2), :], ring2[slot], sem2.at[slot])

    cp1(0, 0).start()
    cp1(1, 1).start()

    x0v = x0[...]
    x1v = x1[...]
    xw01[...] = _dot(x0v, w01[0]).astype(BF16)
    xw10[...] = _dot(x1v, w10[0]).astype(BF16)
    xw21[...] = _dot(x2[...], w21[0]).astype(BF16)
    x1b = x1v.astype(BF16)
    t0 = _dot_tn(e0[...].astype(BF16), x0v.astype(BF16))
    td = _dot_tn(ed1[...].astype(BF16), x1b)
    tu = _dot_tn(eu1[...].astype(BF16), x1b)
    e0v = evs[0]
    s0s[...] = (_dot(e0v * t0, w0[0, 1])
                + _dot(e0v * e0v * t0, w0[0, 2])).astype(BF16)
    ed = evs[1]
    eu = evs[2]
    s1s[0:KEIG, :] = (_dot(ed * td, w1d[0, 0])
                      + _dot(ed * ed * td, w1d[0, 1])).astype(BF16)
    s1s[KEIG:2 * KEIG, :] = (_dot(eu * tu, w1u[0, 0])
                             + _dot(eu * eu * tu, w1u[0, 1])).astype(BF16)

    for i in range(NP1):
        slot = i % 2
        cp1(i, slot).wait()
        inc = ring1[slot][...].astype(BF16)
        y0m[pl.ds(i * BP1, BP1), :] = _dot(inc, xw10[...])
        b = _dot_tn(inc, xw01[pl.ds(i * BP1, BP1), :])
        if i == 0:
            y1acc[...] = b
        else:
            y1acc[...] += b
        nxt = i + 2
        if nxt < NP1:
            cp1(nxt, slot).start()
        else:
            cp2(nxt - NP1, slot).start()

    for i in range(NP2):
        slot = i % 2
        cp2(i, slot).wait()
        inc = ring2[slot][...].astype(BF16)
        y1acc[pl.ds(i * BP2, BP2), :] += _dot(inc, xw21[...])
        nxt = i + 2
        if nxt < NP2:
            cp2(nxt, slot).start()
        else:
            cp1(nxt - NP2, slot).start()

    y0 = (_dot(x0v, w0[0, 0]) + y0m[...]
          + _dot(e0[...].astype(BF16), s0s[...]))
    x0nv = jax.nn.sigmoid(y0)
    x0n[...] = x0nv
    y1 = (_dot(x1v, w1id[0]) + y1acc[...]
          + _dot(ed1[...].astype(BF16), s1s[0:KEIG, :])
          + _dot(eu1[...].astype(BF16), s1s[KEIG:2 * KEIG, :]))
    x1nv = jax.nn.sigmoid(y1)
    xw10[...] = _dot(x1nv, w10[1]).astype(BF16)
    t0b = _dot_tn(e0[...].astype(BF16), x0nv.astype(BF16))
    s0s[...] = (_dot(e0v * t0b, w0[1, 1])
                + _dot(e0v * e0v * t0b, w0[1, 2])).astype(BF16)

    for i in range(NP1):
        slot = i % 2
        cp1(i, slot).wait()
        inc = ring1[slot][...].astype(BF16)
        y0mb[pl.ds(i * BP1, BP1), :] = _dot(inc, xw10[...])
        nxt = i + 2
        if nxt < NP1:
            cp1(nxt, slot).start()

    y0f = (_dot(x0n[...], w0[1, 0]) + y0mb[...]
           + _dot(e0[...].astype(BF16), s0s[...]))
    x0f = jax.nn.sigmoid(y0f)
    out[...] = _dot(x0f, wout[...]) + bout[...]


def _whole(*shape):
    return pl.BlockSpec(shape, lambda: (0,) * len(shape))


def kernel(x_0, x_1, x_2, evals_0, evecs_0, evals_d1, evecs_d1, evals_u1,
           evecs_u1, evals_d2, evecs_d2, evals_u2, evecs_u2, incidence_1,
           incidence_2, W0, W10, W1id, W1d, W1u, W01, W21, W2id, W2d, W2u,
           W12, Wout, bout):
    evs = jnp.stack([evals_0, evals_d1, evals_u1], axis=0).reshape(3, KEIG, 1)

    in_specs = [
        _whole(N0, D), _whole(N1, D), _whole(N2, D),
        _whole(N0, KEIG), _whole(N1, KEIG), _whole(N1, KEIG),
        _whole(3, KEIG, 1),
        _whole(2, 3, D, D),
        _whole(2, D, D), _whole(2, D, D),
        _whole(2, 2, D, D), _whole(2, 2, D, D),
        _whole(2, D, D), _whole(2, D, D),
        _whole(D, NCLS), pl.BlockSpec((NCLS,), lambda: (0,)),
        pl.BlockSpec(memory_space=pltpu.HBM),
        pl.BlockSpec(memory_space=pltpu.HBM),
    ]
    scratch = [
        pltpu.VMEM((N0, D), BF16),
        pltpu.VMEM((N1, D), BF16),
        pltpu.VMEM((N2, D), BF16),
        pltpu.VMEM((N0, D), F32),
        pltpu.VMEM((N1, D), F32),
        pltpu.VMEM((N0, D), F32),
        pltpu.VMEM((KEIG, D), BF16),
        pltpu.VMEM((2 * KEIG, D), BF16),
        pltpu.VMEM((N0, D), F32),
        pltpu.VMEM((BP1, N1), F32),
        pltpu.VMEM((BP1, N1), F32),
        pltpu.VMEM((BP2, N2), F32),
        pltpu.VMEM((BP2, N2), F32),
        pltpu.SemaphoreType.DMA((2,)),
        pltpu.SemaphoreType.DMA((2,)),
    ]
    return pl.pallas_call(
        _net_body,
        in_specs=in_specs,
        out_specs=_whole(N0, NCLS),
        out_shape=jax.ShapeDtypeStruct((N0, NCLS), F32),
        scratch_shapes=scratch,
        compiler_params=pltpu.CompilerParams(
            vmem_limit_bytes=63 * 1024 * 1024,
            fuse_transposed_lhs_in_matmul=True),
    )(x_0, x_1, x_2, evecs_0, evecs_d1, evecs_u1, evs,
      W0, W10, W1id, W1d, W1u, W01, W21, Wout, bout,
      incidence_1, incidence_2)


# speedup vs baseline: 1.0744x; 1.0744x over previous
import jax
import jax.numpy as jnp
from jax.experimental import pallas as pl
from jax.experimental.pallas import tpu as pltpu

F32 = jnp.float32
BF16 = jnp.bfloat16

D = 128
KEIG = 256
NCLS = 9
N0, N1, N2 = 2048, 6144, 4096

BP1 = 256
BP2 = 384
NP1 = N0 // BP1
NP2 = N1 // BP2


def _dot(a, b):
    return jax.lax.dot_general(a, b, (((1,), (0,)), ((), ())),
                               preferred_element_type=F32)


def _dot_tn(a, b):
    return jax.lax.dot_general(a, b, (((0,), (0,)), ((), ())),
                               preferred_element_type=F32)


def _net_body(x0, x1, x2, e0, ed1, eu1, evs,
              w0, w10, w1id, w1d, w1u, w01, w21, wout, bout,
              inc1, inc2,
              out,
              xw01, xw10, xw21, y0m, y1acc, x0n, s0s, s1s, y0mb,
              r1a, r1b, r2a, r2b, sem1, sem2):
    ring1 = (r1a, r1b)
    ring2 = (r2a, r2b)

    def cp1(i, slot):
        return pltpu.make_async_copy(
            inc1.at[pl.ds(i * BP1, BP1), :], ring1[slot], sem1.at[slot])

    def cp2(i, slot):
        return pltpu.make_async_copy(
            inc2.at[pl.ds(i * BP2, BP2), :], ring2[slot], sem2.at[slot])

    cp1(0, 0).start()
    cp1(1, 1).start()

    x0v = x0[...]
    x1v = x1[...]
    xw01[...] = _dot(x0v, w01[0]).astype(BF16)
    xw10[...] = _dot(x1v, w10[0]).astype(BF16)
    xw21[...] = _dot(x2[...], w21[0]).astype(BF16)
    x1b = x1v.astype(BF16)
    t0 = _dot_tn(e0[...].astype(BF16), x0v.astype(BF16))
    td = _dot_tn(ed1[...].astype(BF16), x1b)
    tu = _dot_tn(eu1[...].astype(BF16), x1b)
    e0v = evs[0]
    s0s[...] = (_dot(e0v * t0, w0[0, 1])
                + _dot(e0v * e0v * t0, w0[0, 2])).astype(BF16)
    ed = evs[1]
    eu = evs[2]
    s1s[0:KEIG, :] = (_dot(ed * td, w1d[0, 0])
                      + _dot(ed * ed * td, w1d[0, 1])).astype(BF16)
    s1s[KEIG:2 * KEIG, :] = (_dot(eu * tu, w1u[0, 0])
                             + _dot(eu * eu * tu, w1u[0, 1])).astype(BF16)

    for i in range(NP1):
        slot = i % 2
        cp1(i, slot).wait()
        inc = ring1[slot][...].astype(BF16)
        y0m[pl.ds(i * BP1, BP1), :] = _dot(inc, xw10[...])
        b = _dot_tn(inc, xw01[pl.ds(i * BP1, BP1), :])
        if i == 0:
            y1acc[...] = b
        else:
            y1acc[...] += b
        nxt = i + 2
        if nxt < NP1:
            cp1(nxt, slot).start()
        else:
            cp2(nxt - NP1, slot).start()

    for i in range(NP2):
        slot = i % 2
        cp2(i, slot).wait()
        inc = ring2[slot][...].astype(BF16)
        y1acc[pl.ds(i * BP2, BP2), :] += _dot(inc, xw21[...])
        nxt = i + 2
        if nxt < NP2:
            cp2(nxt, slot).start()
        else:
            cp1(nxt - NP2, slot).start()

    y0 = (_dot(x0v, w0[0, 0]) + y0m[...]
          + _dot(e0[...].astype(BF16), s0s[...]))
    x0nv = jax.nn.sigmoid(y0)
    x0n[...] = x0nv
    y1 = (_dot(x1v, w1id[0]) + y1acc[...]
          + _dot(ed1[...].astype(BF16), s1s[0:KEIG, :])
          + _dot(eu1[...].astype(BF16), s1s[KEIG:2 * KEIG, :]))
    x1nv = jax.nn.sigmoid(y1)
    xw10[...] = _dot(x1nv, w10[1]).astype(BF16)
    t0b = _dot_tn(e0[...].astype(BF16), x0nv.astype(BF16))
    s0s[...] = (_dot(e0v * t0b, w0[1, 1])
                + _dot(e0v * e0v * t0b, w0[1, 2])).astype(BF16)

    for i in range(NP1):
        slot = i % 2
        cp1(i, slot).wait()
        inc = ring1[slot][...].astype(BF16)
        y0mb[pl.ds(i * BP1, BP1), :] = _dot(inc, xw10[...])
        nxt = i + 2
        if nxt < NP1:
            cp1(nxt, slot).start()

    y0f = (_dot(x0n[...], w0[1, 0]) + y0mb[...]
           + _dot(e0[...].astype(BF16), s0s[...]))
    x0f = jax.nn.sigmoid(y0f)
    out[...] = _dot(x0f, wout[...]) + bout[...]


def _whole(*shape):
    return pl.BlockSpec(shape, lambda: (0,) * len(shape))


def kernel(x_0, x_1, x_2, evals_0, evecs_0, evals_d1, evecs_d1, evals_u1,
           evecs_u1, evals_d2, evecs_d2, evals_u2, evecs_u2, incidence_1,
           incidence_2, W0, W10, W1id, W1d, W1u, W01, W21, W2id, W2d, W2u,
           W12, Wout, bout):
    evs = jnp.stack([evals_0, evals_d1, evals_u1], axis=0).reshape(3, KEIG, 1)

    in_specs = [
        _whole(N0, D), _whole(N1, D), _whole(N2, D),
        _whole(N0, KEIG), _whole(N1, KEIG), _whole(N1, KEIG),
        _whole(3, KEIG, 1),
        _whole(2, 3, D, D),
        _whole(2, D, D), _whole(2, D, D),
        _whole(2, 2, D, D), _whole(2, 2, D, D),
        _whole(2, D, D), _whole(2, D, D),
        _whole(D, NCLS), pl.BlockSpec((NCLS,), lambda: (0,)),
        pl.BlockSpec(memory_space=pltpu.HBM),
        pl.BlockSpec(memory_space=pltpu.HBM),
    ]
    scratch = [
        pltpu.VMEM((N0, D), BF16),
        pltpu.VMEM((N1, D), BF16),
        pltpu.VMEM((N2, D), BF16),
        pltpu.VMEM((N0, D), F32),
        pltpu.VMEM((N1, D), F32),
        pltpu.VMEM((N0, D), F32),
        pltpu.VMEM((KEIG, D), BF16),
        pltpu.VMEM((2 * KEIG, D), BF16),
        pltpu.VMEM((N0, D), F32),
        pltpu.VMEM((BP1, N1), F32),
        pltpu.VMEM((BP1, N1), F32),
        pltpu.VMEM((BP2, N2), F32),
        pltpu.VMEM((BP2, N2), F32),
        pltpu.SemaphoreType.DMA((2,)),
        pltpu.SemaphoreType.DMA((2,)),
    ]
    return pl.pallas_call(
        _net_body,
        in_specs=in_specs,
        out_specs=_whole(N0, NCLS),
        out_shape=jax.ShapeDtypeStruct((N0, NCLS), F32),
        scratch_shapes=scratch,
        compiler_params=pltpu.CompilerParams(
            vmem_limit_bytes=63 * 1024 * 1024),
    )(x_0, x_1, x_2, evecs_0, evecs_d1, evecs_u1, evs,
      W0, W10, W1id, W1d, W1u, W01, W21, Wout, bout,
      incidence_1, incidence_2)
